# baseline (speedup 1.0000x reference)
import jax
import jax.numpy as jnp
from jax.experimental import pallas as pl

_B = 64
_C = 3
_H = 224
_W = 224
_P = 14
_GH = 16
_GW = 16
_D = 768
_MAXT = 10
_CROP = 0
_EFF_W = _W - 2 * _CROP
_PATCH_H = _H / _GH
_PATCH_W = _EFF_W / _GW
_NP = _GH * _GW
_K = _C * _P * _P
_G = 8


def _fused(x_ref, boxes_ref, wp_ref, wa_ref, ba_ref, cls_ref, obj_ref, att_ref):
    x = x_ref[...].reshape(_G * _NP, _K)
    pt = jnp.dot(x, wp_ref[...], preferred_element_type=jnp.float32)

    wa = wa_ref[...]
    logits_all = jax.lax.dot_general(
        wa, pt, (((1,), (1,)), ((), ())), preferred_element_type=jnp.float32
    ) + ba_ref[0, 0]

    for g in range(_G):
        pt_g = pt[g * _NP:(g + 1) * _NP, :]
        cls_ref[g] = jnp.mean(pt_g, axis=0, keepdims=True)
        logits = logits_all[:, g * _NP:(g + 1) * _NP]

        bx = boxes_ref[g]
        x0 = jnp.clip(bx[:, 0:1] * _W - _CROP, 0.0, float(_EFF_W))
        y0 = jnp.clip(bx[:, 1:2] * _H, 0.0, float(_H))
        x1 = jnp.clip(bx[:, 2:3] * _W - _CROP, 0.0, float(_EFF_W))
        y1 = jnp.clip(bx[:, 3:4] * _H, 0.0, float(_H))

        x0i = jnp.clip(jnp.floor(x0 / _PATCH_W).astype(jnp.int32), 0, _GW - 1)
        y0i = jnp.clip(jnp.floor(y0 / _PATCH_H).astype(jnp.int32), 0, _GH - 1)
        x1i = jnp.clip(jnp.ceil(x1 / _PATCH_W).astype(jnp.int32), x0i + 1, _GW)
        y1i = jnp.clip(jnp.ceil(y1 / _PATCH_H).astype(jnp.int32), y0i + 1, _GH)

        p = jax.lax.broadcasted_iota(jnp.int32, (_MAXT, _NP), 1)
        py = p // _GW
        px = p % _GW
        mask = (py >= y0i) & (py < y1i) & (px >= x0i) & (px < x1i)

        ml = jnp.where(mask, jnp.broadcast_to(logits, (_MAXT, _NP)), -jnp.inf)
        mx = jnp.max(ml, axis=1, keepdims=True)
        ew = jnp.exp(ml - mx)
        sw = jnp.sum(ew, axis=1, keepdims=True)
        wts = ew / sw

        att_ref[g] = wts
        obj_ref[g] = jnp.dot(wts, pt_g, preferred_element_type=jnp.float32)


_NCHUNK = 4
_CB = _B // _NCHUNK


def _run_chunk(images_c, boxes_c, W_patch, wa, ba):
    x = images_c.reshape(_CB, _C, _GH, _P, _GW, _P)
    x = jnp.transpose(x, (0, 2, 4, 1, 3, 5)).reshape(_CB, _NP, _K)
    return pl.pallas_call(
        _fused,
        grid=(_CB // _G,),
        in_specs=[
            pl.BlockSpec((_G, _NP, _K), lambda b: (b, 0, 0)),
            pl.BlockSpec((_G, _MAXT, 4), lambda b: (b, 0, 0)),
            pl.BlockSpec((_K, _D), lambda b: (0, 0)),
            pl.BlockSpec((1, _D), lambda b: (0, 0)),
            pl.BlockSpec((1, 1), lambda b: (0, 0)),
        ],
        out_specs=[
            pl.BlockSpec((_G, 1, _D), lambda b: (b, 0, 0)),
            pl.BlockSpec((_G, _MAXT, _D), lambda b: (b, 0, 0)),
            pl.BlockSpec((_G, _MAXT, _NP), lambda b: (b, 0, 0)),
        ],
        out_shape=[
            jax.ShapeDtypeStruct((_CB, 1, _D), jnp.float32),
            jax.ShapeDtypeStruct((_CB, _MAXT, _D), jnp.float32),
            jax.ShapeDtypeStruct((_CB, _MAXT, _NP), jnp.float32),
        ],
    )(x, boxes_c, W_patch, wa, ba)


def kernel(images, boxes, scores, W_patch, W_att, b_att):
    wa = W_att.reshape(1, _D)
    ba = b_att.reshape(1, 1)

    cls_parts, obj_parts, att_parts = [], [], []
    for c in range(_NCHUNK):
        s = c * _CB
        cls_c, obj_c, att_c = _run_chunk(
            images[s:s + _CB], boxes[s:s + _CB], W_patch, wa, ba)
        cls_parts.append(cls_c)
        obj_parts.append(obj_c)
        att_parts.append(att_c)

    cls_tokens = jnp.concatenate(cls_parts, axis=0).reshape(_B, _D)
    object_tokens = jnp.concatenate(obj_parts, axis=0)
    attention_maps = jnp.concatenate(att_parts, axis=0)

    object_mask = jnp.ones((_B, _MAXT), dtype=bool)
    return (cls_tokens, object_tokens, object_mask, boxes, scores, attention_maps)


# speedup vs baseline: 2.1183x; 2.1183x over previous
import jax
import jax.numpy as jnp
from jax.experimental import pallas as pl

_B = 64
_C = 3
_H = 224
_W = 224
_P = 14
_GH = 16
_GW = 16
_D = 768
_MAXT = 10
_CROP = 0
_EFF_W = _W - 2 * _CROP
_PATCH_H = _H / _GH
_PATCH_W = _EFF_W / _GW
_NP = _GH * _GW
_K = _C * _P * _P
_G = 8
_R = _G * _MAXT
_Q = _G * _NP


def _fused(x_ref, boxes_ref, wp_ref, wa_ref, ba_ref, cls_ref, obj_ref, att_ref):
    x = x_ref[...].reshape(_Q, _K)
    pt = jnp.dot(x, wp_ref[...], preferred_element_type=jnp.float32)

    qg = jax.lax.broadcasted_iota(jnp.int32, (_G, _Q), 1) // _NP
    gsel = jax.lax.broadcasted_iota(jnp.int32, (_G, _Q), 0)
    cmat = jnp.where(qg == gsel, 1.0 / _NP, 0.0)
    cls_ref[...] = jnp.dot(cmat, pt, preferred_element_type=jnp.float32)

    wa = wa_ref[...]
    logits = jax.lax.dot_general(
        wa, pt, (((1,), (1,)), ((), ())), preferred_element_type=jnp.float32
    ) + ba_ref[0, 0]

    bx = boxes_ref[...]
    x0 = jnp.clip(bx[:, 0:1] * _W - _CROP, 0.0, float(_EFF_W))
    y0 = jnp.clip(bx[:, 1:2] * _H, 0.0, float(_H))
    x1 = jnp.clip(bx[:, 2:3] * _W - _CROP, 0.0, float(_EFF_W))
    y1 = jnp.clip(bx[:, 3:4] * _H, 0.0, float(_H))

    x0i = jnp.clip(jnp.floor(x0 / _PATCH_W).astype(jnp.int32), 0, _GW - 1)
    y0i = jnp.clip(jnp.floor(y0 / _PATCH_H).astype(jnp.int32), 0, _GH - 1)
    x1i = jnp.clip(jnp.ceil(x1 / _PATCH_W).astype(jnp.int32), x0i + 1, _GW)
    y1i = jnp.clip(jnp.ceil(y1 / _PATCH_H).astype(jnp.int32), y0i + 1, _GH)

    q = jax.lax.broadcasted_iota(jnp.int32, (_R, _Q), 1)
    r = jax.lax.broadcasted_iota(jnp.int32, (_R, _Q), 0)
    p = q % _NP
    py = p // _GW
    px = p % _GW
    same_img = (q // _NP) == (r // _MAXT)
    mask = (same_img & (py >= y0i) & (py < y1i)
            & (px >= x0i) & (px < x1i))

    ml = jnp.where(mask, jnp.broadcast_to(logits, (_R, _Q)), -jnp.inf)
    mx = jnp.max(ml, axis=1, keepdims=True)
    ew = jnp.exp(ml - mx)
    sw = jnp.sum(ew, axis=1, keepdims=True)
    wts = ew / sw

    att = wts[:, 0:_NP]
    for g in range(1, _G):
        att = att + wts[:, g * _NP:(g + 1) * _NP]
    att_ref[...] = att

    obj_ref[...] = jnp.dot(wts, pt, preferred_element_type=jnp.float32)


def kernel(images, boxes, scores, W_patch, W_att, b_att):
    x = images.reshape(_B, _C, _GH, _P, _GW, _P)
    x = jnp.transpose(x, (0, 2, 4, 1, 3, 5)).reshape(_B, _NP, _K)
    boxes_flat = boxes.reshape(_B * _MAXT, 4)
    wa = W_att.reshape(1, _D)
    ba = b_att.reshape(1, 1)

    cls_tokens, object_tokens, attention_maps = pl.pallas_call(
        _fused,
        grid=(_B // _G,),
        in_specs=[
            pl.BlockSpec((_G, _NP, _K), lambda b: (b, 0, 0)),
            pl.BlockSpec((_R, 4), lambda b: (b, 0)),
            pl.BlockSpec((_K, _D), lambda b: (0, 0)),
            pl.BlockSpec((1, _D), lambda b: (0, 0)),
            pl.BlockSpec((1, 1), lambda b: (0, 0)),
        ],
        out_specs=[
            pl.BlockSpec((_G, _D), lambda b: (b, 0)),
            pl.BlockSpec((_R, _D), lambda b: (b, 0)),
            pl.BlockSpec((_R, _NP), lambda b: (b, 0)),
        ],
        out_shape=[
            jax.ShapeDtypeStruct((_B, _D), jnp.float32),
            jax.ShapeDtypeStruct((_B * _MAXT, _D), jnp.float32),
            jax.ShapeDtypeStruct((_B * _MAXT, _NP), jnp.float32),
        ],
    )(x, boxes_flat, W_patch, wa, ba)

    object_tokens = object_tokens.reshape(_B, _MAXT, _D)
    attention_maps = attention_maps.reshape(_B, _MAXT, _NP)
    object_mask = jnp.ones((_B, _MAXT), dtype=bool)
    return (cls_tokens, object_tokens, object_mask, boxes, scores, attention_maps)


# speedup vs baseline: 2.2191x; 1.0476x over previous
import jax
import jax.numpy as jnp
from jax.experimental import pallas as pl

_B = 64
_C = 3
_H = 224
_W = 224
_P = 14
_GH = 16
_GW = 16
_D = 768
_MAXT = 10
_CROP = 0
_EFF_W = _W - 2 * _CROP
_PATCH_H = _H / _GH
_PATCH_W = _EFF_W / _GW
_NP = _GH * _GW
_K = _C * _P * _P
_G = 8
_R = _G * _MAXT
_Q = _G * _NP


def _fused(x_ref, boxes_ref, wp_ref, wa_ref, ba_ref, cls_ref, obj_ref, att_ref):
    x = x_ref[...].reshape(_Q, _K)
    pt = jnp.dot(x, wp_ref[...], preferred_element_type=jnp.float32)

    qg = jax.lax.broadcasted_iota(jnp.int32, (_G, _Q), 1) // _NP
    gsel = jax.lax.broadcasted_iota(jnp.int32, (_G, _Q), 0)
    cmat = jnp.where(qg == gsel, 1.0 / _NP, 0.0)
    cls_ref[...] = jnp.dot(cmat, pt, preferred_element_type=jnp.float32)

    wa = wa_ref[...]
    logits = jax.lax.dot_general(
        wa, pt, (((1,), (1,)), ((), ())), preferred_element_type=jnp.float32
    ) + ba_ref[0, 0]

    bx = boxes_ref[...]
    x0 = jnp.clip(bx[:, 0:1] * _W - _CROP, 0.0, float(_EFF_W))
    y0 = jnp.clip(bx[:, 1:2] * _H, 0.0, float(_H))
    x1 = jnp.clip(bx[:, 2:3] * _W - _CROP, 0.0, float(_EFF_W))
    y1 = jnp.clip(bx[:, 3:4] * _H, 0.0, float(_H))

    x0i = jnp.clip(jnp.floor(x0 / _PATCH_W).astype(jnp.int32), 0, _GW - 1)
    y0i = jnp.clip(jnp.floor(y0 / _PATCH_H).astype(jnp.int32), 0, _GH - 1)
    x1i = jnp.clip(jnp.ceil(x1 / _PATCH_W).astype(jnp.int32), x0i + 1, _GW)
    y1i = jnp.clip(jnp.ceil(y1 / _PATCH_H).astype(jnp.int32), y0i + 1, _GH)

    q = jax.lax.broadcasted_iota(jnp.int32, (_R, _Q), 1)
    r = jax.lax.broadcasted_iota(jnp.int32, (_R, _Q), 0)
    p = q % _NP
    py = p // _GW
    px = p % _GW
    same_img = (q // _NP) == (r // _MAXT)
    mask = (same_img & (py >= y0i) & (py < y1i)
            & (px >= x0i) & (px < x1i))

    ml = jnp.where(mask, jnp.broadcast_to(logits, (_R, _Q)), -jnp.inf)
    mx = jnp.max(ml, axis=1, keepdims=True)
    ew = jnp.exp(ml - mx)
    sw = jnp.sum(ew, axis=1, keepdims=True)
    wts = ew / sw

    att = wts[:, 0:_NP]
    for g in range(1, _G):
        att = att + wts[:, g * _NP:(g + 1) * _NP]
    att_ref[...] = att

    obj_ref[...] = jnp.dot(wts, pt, preferred_element_type=jnp.float32)


def kernel(images, boxes, scores, W_patch, W_att, b_att):
    x = images.astype(jnp.bfloat16).reshape(_B, _C, _GH, _P, _GW, _P)
    x = jnp.transpose(x, (0, 2, 4, 1, 3, 5)).reshape(_B, _NP, _K)
    W_patch = W_patch.astype(jnp.bfloat16)
    boxes_flat = boxes.reshape(_B * _MAXT, 4)
    wa = W_att.reshape(1, _D)
    ba = b_att.reshape(1, 1)

    cls_tokens, object_tokens, attention_maps = pl.pallas_call(
        _fused,
        grid=(_B // _G,),
        in_specs=[
            pl.BlockSpec((_G, _NP, _K), lambda b: (b, 0, 0)),
            pl.BlockSpec((_R, 4), lambda b: (b, 0)),
            pl.BlockSpec((_K, _D), lambda b: (0, 0)),
            pl.BlockSpec((1, _D), lambda b: (0, 0)),
            pl.BlockSpec((1, 1), lambda b: (0, 0)),
        ],
        out_specs=[
            pl.BlockSpec((_G, _D), lambda b: (b, 0)),
            pl.BlockSpec((_R, _D), lambda b: (b, 0)),
            pl.BlockSpec((_R, _NP), lambda b: (b, 0)),
        ],
        out_shape=[
            jax.ShapeDtypeStruct((_B, _D), jnp.float32),
            jax.ShapeDtypeStruct((_B * _MAXT, _D), jnp.float32),
            jax.ShapeDtypeStruct((_B * _MAXT, _NP), jnp.float32),
        ],
    )(x, boxes_flat, W_patch, wa, ba)

    object_tokens = object_tokens.reshape(_B, _MAXT, _D)
    attention_maps = attention_maps.reshape(_B, _MAXT, _NP)
    object_mask = jnp.ones((_B, _MAXT), dtype=bool)
    return (cls_tokens, object_tokens, object_mask, boxes, scores, attention_maps)
